# baseline (speedup 1.0000x reference)
import functools

import jax
import jax.numpy as jnp
from jax import lax
from jax.experimental import pallas as pl
from jax.experimental.pallas import tpu as pltpu
from jax.experimental.pallas import tpu_sc as plsc

E = 800000
CIN = 11
H = 64
B = 128
NC = 2
NS = 16
NW = NC * NS
CPW = 200
EP = NW * CPW * B
NCH = EP // B
BT = 1024


SB = 20
NBUF = 3


@functools.lru_cache(maxsize=None)
def _make_gather(C):
    mesh = plsc.VectorSubcoreMesh(core_axis_name="c", subcore_axis_name="s")

    @functools.partial(
        pl.kernel,
        out_type=jax.ShapeDtypeStruct((4, EP, C), jnp.float32),
        mesh=mesh,
        compiler_params=pltpu.CompilerParams(use_tc_tiling_on_sc=False),
        scratch_types=[
            pltpu.VMEM((SB, 4, B), jnp.int32),
            pltpu.VMEM((NBUF, 4, B, C), jnp.float32),
        ] + [pltpu.SemaphoreType.DMA] * (2 * NBUF),
    )
    def gather_k(nb_hbm, h_hbm, out_hbm, idx_v, gath_v, *sems):
        gsems, wsems = sems[:NBUF], sems[NBUF:]
        wid = lax.axis_index("c") * NS + lax.axis_index("s")
        base = wid * CPW

        def super_body(t, carry):
            ch0 = base + t * SB
            pltpu.sync_copy(nb_hbm.at[pl.ds(ch0, SB)], idx_v)
            gcp, wcp = {}, {}

            def start_stores(kk):
                ss = kk % NBUF
                for c in gcp[kk]:
                    c.wait()
                wcp[kk] = [
                    pltpu.async_copy(gath_v.at[ss, j],
                                     out_hbm.at[j, pl.ds((ch0 + kk) * B, B)],
                                     wsems[ss])
                    for j in range(4)
                ]

            for k in range(SB):
                s = k % NBUF
                if k >= NBUF:
                    for c in wcp[k - NBUF]:
                        c.wait()
                gcp[k] = [
                    pltpu.async_copy(h_hbm.at[idx_v.at[k, j]],
                                     gath_v.at[s, j], gsems[s])
                    for j in range(4)
                ]
                if k >= 2:
                    start_stores(k - 2)
            start_stores(SB - 2)
            start_stores(SB - 1)
            for k in range(SB - NBUF, SB):
                for c in wcp[k]:
                    c.wait()
            return carry

        lax.fori_loop(0, CPW // SB, super_body, None)

    return gather_k


def _conv(hb, g_ref, w_ref, cin):
    n0, n1, n2, n3 = g_ref[0], g_ref[1], g_ref[2], g_ref[3]
    feats = (hb, n0 + n2, jnp.abs(n0 - n2), n1 + n3, jnp.abs(n1 - n3))
    z = None
    for k, f in enumerate(feats):
        zk = jnp.dot(f, w_ref[k * cin:(k + 1) * cin, :],
                     preferred_element_type=jnp.float32)
        z = zk if z is None else z + zk
    return z


def _ln_relu(z, p_ref):
    z = z + p_ref[0][None, :]
    m = jnp.mean(z, axis=1, keepdims=True)
    zc = z - m
    v = jnp.mean(zc * zc, axis=1, keepdims=True)
    zn = zc * lax.rsqrt(v + 1e-5) * p_ref[1][None, :] + p_ref[2][None, :]
    return jnp.maximum(zn, 0.0)


def _tc0_body(h_ref, g_ref, w_ref, p_ref, o_ref):
    o_ref[...] = _ln_relu(_conv(h_ref[...], g_ref, w_ref, 16), p_ref)


def _tc_mid_body(h_ref, g_ref, w_ref, p_ref, o_ref):
    hb = h_ref[...]
    o_ref[...] = _ln_relu(_conv(hb, g_ref, w_ref, 64), p_ref) + hb


def _tc_last_body(h_ref, g_ref, w_ref, p_ref, cw1_ref, hp_ref, o_ref):
    hb = h_ref[...]
    a = _ln_relu(_conv(hb, g_ref, w_ref, 64), p_ref) + hb
    t = jnp.maximum(
        jnp.dot(a, cw1_ref[...], preferred_element_type=jnp.float32)
        + hp_ref[0, :32][None, :], 0.0)
    o_ref[...] = jnp.sum(t * hp_ref[1, :32][None, :], axis=1) + hp_ref[2, 0]


def _mk_tc(body, cin, out_shape, out_spec, extra_specs=()):
    return pl.pallas_call(
        body,
        grid=(EP // BT,),
        in_specs=[
            pl.BlockSpec((BT, cin), lambda i: (i, 0)),
            pl.BlockSpec((4, BT, cin), lambda i: (0, i, 0)),
            pl.BlockSpec((5 * cin, H), lambda i: (0, 0)),
            pl.BlockSpec((8, H), lambda i: (0, 0)),
            *extra_specs,
        ],
        out_specs=out_spec,
        out_shape=out_shape,
    )


_TC0 = _mk_tc(_tc0_body, 16,
              jax.ShapeDtypeStruct((EP, H), jnp.float32),
              pl.BlockSpec((BT, H), lambda i: (i, 0)))
_TCM = _mk_tc(_tc_mid_body, 64,
              jax.ShapeDtypeStruct((EP, H), jnp.float32),
              pl.BlockSpec((BT, H), lambda i: (i, 0)))
_TCL = _mk_tc(_tc_last_body, 64,
              jax.ShapeDtypeStruct((EP,), jnp.float32),
              pl.BlockSpec((BT,), lambda i: (i,)),
              extra_specs=(pl.BlockSpec((H, 32), lambda i: (0, 0)),
                           pl.BlockSpec((8, H), lambda i: (0, 0))))


def kernel(x, neighbors, W0, b0, g0, be0, W1, b1, g1, be1,
           W2, b2, g2, be2, W3, b3, g3, be3, cW1, cb1, cW2, cb2):
    xp = jnp.pad(x, ((0, EP - E), (0, 16 - CIN)))
    nbp = jnp.pad(neighbors, ((0, EP - E), (0, 0)))
    nb3 = nbp.T.reshape(4, NCH, B).transpose(1, 0, 2)

    w0p = jnp.zeros((80, H), jnp.float32)
    for k in range(5):
        w0p = w0p.at[k * 16:k * 16 + CIN].set(W0[k * CIN:(k + 1) * CIN])

    def pack(b, g, be):
        return jnp.concatenate(
            [b[None], g[None], be[None], jnp.zeros((5, H), jnp.float32)], 0)

    hp = jnp.zeros((8, H), jnp.float32)
    hp = hp.at[0, :32].set(cb1)
    hp = hp.at[1, :32].set(cW2[:, 0])
    hp = hp.at[2, 0].set(cb2[0])

    g16, g64 = _make_gather(16), _make_gather(64)
    h = _TC0(xp, g16(nb3, xp), w0p, pack(b0, g0, be0))
    h = _TCM(h, g64(nb3, h), W1, pack(b1, g1, be1))
    h = _TCM(h, g64(nb3, h), W2, pack(b2, g2, be2))
    out = _TCL(h, g64(nb3, h), W3, pack(b3, g3, be3), cW1, hp)
    return out[:E]


# speedup vs baseline: 1.1208x; 1.1208x over previous
import functools

import jax
import jax.numpy as jnp
from jax import lax
from jax.experimental import pallas as pl
from jax.experimental.pallas import tpu as pltpu
from jax.experimental.pallas import tpu_sc as plsc

E = 800000
CIN = 11
H = 64
B = 128
NC = 2
NS = 16
NW = NC * NS
CPW = 200
EP = NW * CPW * B
NCH = EP // B
BT = 1024
SB = 10
NBUF = 3


@functools.lru_cache(maxsize=None)
def _make_gather(C):
    n_planes = 4 if C == 64 else 1
    out_shape = ((4, EP // 2, 128) if C == 64 else (EP // 2, 128))
    mesh = plsc.VectorSubcoreMesh(core_axis_name="c", subcore_axis_name="s")

    @functools.partial(
        pl.kernel,
        out_type=jax.ShapeDtypeStruct(out_shape, jnp.float32),
        mesh=mesh,
        compiler_params=pltpu.CompilerParams(use_tc_tiling_on_sc=False),
        scratch_types=[
            pltpu.VMEM((SB * 4, B), jnp.int32),
            pltpu.VMEM((NBUF, 8, B // 2, C), jnp.float32),
        ] + [pltpu.SemaphoreType.DMA] * (2 * NBUF),
    )
    def gather_k(nb_hbm, h_hbm, out_hbm, idx_v, gath_v, *sems):
        gsems, wsems = sems[:NBUF], sems[NBUF:]
        wid = lax.axis_index("c") * NS + lax.axis_index("s")
        base = wid * CPW

        def super_body(t, carry):
            ch0 = base + t * SB
            pltpu.sync_copy(nb_hbm.at[pl.ds(ch0 * 4, SB * 4)], idx_v)
            gcp, wcp = {}, {}

            def dst(kk, j, par):
                row0 = (ch0 + kk) * (B // 2)
                if C == 64:
                    return out_hbm.at[j, pl.ds(row0, B // 2),
                                      pl.ds(par * 64, 64)]
                return out_hbm.at[pl.ds(row0, B // 2),
                                  pl.ds(par * 64 + j * 16, 16)]

            def start_stores(kk):
                ss = kk % NBUF
                for c in gcp[kk]:
                    c.wait()
                wcp[kk] = [
                    pltpu.async_copy(gath_v.at[ss, par * 4 + j],
                                     dst(kk, j, par), wsems[ss])
                    for par in range(2) for j in range(4)
                ]

            for k in range(SB):
                s = k % NBUF
                if k >= NBUF:
                    for c in wcp[k - NBUF]:
                        c.wait()
                gcp[k] = [
                    pltpu.async_copy(
                        h_hbm.at[idx_v.at[k * 4 + j, pl.ds(par * 64, 64)]],
                        gath_v.at[s, par * 4 + j], gsems[s])
                    for par in range(2) for j in range(4)
                ]
                if k >= 2:
                    start_stores(k - 2)
            start_stores(SB - 2)
            start_stores(SB - 1)
            for k in range(SB - NBUF, SB):
                for c in wcp[k]:
                    c.wait()
            return carry

        lax.fori_loop(0, CPW // SB, super_body, None)

    return gather_k


def _ln_relu_pair(z, p_ref):
    halves = []
    for h0 in (0, 64):
        zz = z[:, h0:h0 + 64] + p_ref[0][None, :]
        m = jnp.mean(zz, axis=1, keepdims=True)
        zc = zz - m
        v = jnp.mean(zc * zc, axis=1, keepdims=True)
        halves.append(jnp.maximum(
            zc * lax.rsqrt(v + 1e-5) * p_ref[1][None, :]
            + p_ref[2][None, :], 0.0))
    return jnp.concatenate(halves, axis=1)


def _mid_act(h_ref, g_ref, w_ref, p_ref):
    hb = h_ref[...]
    g0, g1, g2, g3 = g_ref[0], g_ref[1], g_ref[2], g_ref[3]
    f = jnp.concatenate(
        [hb, g0 + g2, jnp.abs(g0 - g2), g1 + g3, jnp.abs(g1 - g3)], axis=1)
    z = jnp.dot(f, w_ref[...], preferred_element_type=jnp.float32)
    return _ln_relu_pair(z, p_ref) + hb


def _tc0_body(x_ref, g_ref, w_ref, p_ref, o_ref):
    xb = x_ref[...]
    gx = g_ref[...]
    pieces = [xb]
    for h0 in (0, 64):
        n0 = gx[:, h0:h0 + 16]
        n1 = gx[:, h0 + 16:h0 + 32]
        n2 = gx[:, h0 + 32:h0 + 48]
        n3 = gx[:, h0 + 48:h0 + 64]
        pieces += [n0 + n2, jnp.abs(n0 - n2), n1 + n3, jnp.abs(n1 - n3)]
    f = jnp.concatenate(pieces, axis=1)
    z = jnp.dot(f, w_ref[...], preferred_element_type=jnp.float32)
    o_ref[...] = _ln_relu_pair(z, p_ref)


def _tc_mid_body(h_ref, g_ref, w_ref, p_ref, o_ref):
    o_ref[...] = _mid_act(h_ref, g_ref, w_ref, p_ref)


def _tc_last_body(h_ref, g_ref, w_ref, p_ref, cw1_ref, hp_ref, o_ref):
    a = _mid_act(h_ref, g_ref, w_ref, p_ref)
    t = jnp.maximum(
        jnp.dot(a, cw1_ref[...], preferred_element_type=jnp.float32)
        + hp_ref[0][None, :], 0.0)
    w2 = hp_ref[1, :32][None, :]
    o_l = jnp.sum(t[:, :32] * w2, axis=1) + hp_ref[2, 0]
    o_r = jnp.sum(t[:, 32:64] * w2, axis=1) + hp_ref[2, 0]
    o_ref[...] = jnp.concatenate([o_l[:, None], o_r[:, None]], axis=1)


_R = BT // 2


def _mk_tc(body, g_spec, kw, out_shape, out_spec, extra_specs=()):
    return pl.pallas_call(
        body,
        grid=(EP // BT,),
        in_specs=[
            pl.BlockSpec((_R, 128), lambda i: (i, 0)),
            g_spec,
            pl.BlockSpec((kw, 128), lambda i: (0, 0)),
            pl.BlockSpec((8, H), lambda i: (0, 0)),
            *extra_specs,
        ],
        out_specs=out_spec,
        out_shape=out_shape,
    )


_TC0 = _mk_tc(_tc0_body,
              pl.BlockSpec((_R, 128), lambda i: (i, 0)), 256,
              jax.ShapeDtypeStruct((EP // 2, 128), jnp.float32),
              pl.BlockSpec((_R, 128), lambda i: (i, 0)))
_TCM = _mk_tc(_tc_mid_body,
              pl.BlockSpec((4, _R, 128), lambda i: (0, i, 0)), 640,
              jax.ShapeDtypeStruct((EP // 2, 128), jnp.float32),
              pl.BlockSpec((_R, 128), lambda i: (i, 0)))
_TCL = _mk_tc(_tc_last_body,
              pl.BlockSpec((4, _R, 128), lambda i: (0, i, 0)), 640,
              jax.ShapeDtypeStruct((EP // 2, 2), jnp.float32),
              pl.BlockSpec((_R, 2), lambda i: (i, 0)),
              extra_specs=(pl.BlockSpec((128, H), lambda i: (0, 0)),
                           pl.BlockSpec((8, H), lambda i: (0, 0))))


def _lr(Wk, side):
    z = jnp.zeros_like(Wk)
    return jnp.concatenate([Wk, z] if side == 0 else [z, Wk], axis=1)


def _wbig_mid(W):
    blocks = []
    for k in range(5):
        Wk = W[k * 64:(k + 1) * 64]
        blocks += [_lr(Wk, 0), _lr(Wk, 1)]
    return jnp.concatenate(blocks, axis=0)


def _wbig0(W0):
    wx = jnp.zeros((64, H), jnp.float32).at[:CIN].set(W0[:CIN])
    combo = [jnp.zeros((16, H), jnp.float32).at[:CIN].set(
        W0[k * CIN:(k + 1) * CIN]) for k in range(1, 5)]
    blocks = [_lr(wx, 0), _lr(wx, 1)]
    blocks += [_lr(c, 0) for c in combo]
    blocks += [_lr(c, 1) for c in combo]
    return jnp.concatenate(blocks, axis=0)


def kernel(x, neighbors, W0, b0, g0, be0, W1, b1, g1, be1,
           W2, b2, g2, be2, W3, b3, g3, be3, cW1, cb1, cW2, cb2):
    x64 = jnp.pad(x, ((0, EP - E), (0, 64 - CIN))).reshape(EP // 2, 128)
    x16 = jnp.pad(x, ((0, EP - E), (0, 16 - CIN)))
    nbi = jnp.pad(neighbors, ((0, EP - E), (0, 0)))
    nbq = (nbi.reshape(NCH, 64, 2, 4).transpose(0, 3, 2, 1)
           .reshape(NCH * 4, B))

    def pack(b, g, be):
        return jnp.concatenate(
            [b[None], g[None], be[None], jnp.zeros((5, H), jnp.float32)], 0)

    cw1b = jnp.zeros((128, 64), jnp.float32)
    cw1b = cw1b.at[:64, :32].set(cW1).at[64:, 32:].set(cW1)
    hp = jnp.zeros((8, H), jnp.float32)
    hp = hp.at[0, :32].set(cb1).at[0, 32:].set(cb1)
    hp = hp.at[1, :32].set(cW2[:, 0])
    hp = hp.at[2, 0].set(cb2[0])

    g16, g64 = _make_gather(16), _make_gather(64)
    gx = g16(nbq, x16)
    h = _TC0(x64, gx, _wbig0(W0), pack(b0, g0, be0))
    for (W, b, g, be) in ((W1, b1, g1, be1), (W2, b2, g2, be2)):
        gp = g64(nbq, h.reshape(EP, 64))
        h = _TCM(h, gp, _wbig_mid(W), pack(b, g, be))
    gp = g64(nbq, h.reshape(EP, 64))
    out = _TCL(h, gp, _wbig_mid(W3), pack(b3, g3, be3), cw1b, hp)
    return out.reshape(EP)[:E]
